# Initial kernel scaffold; baseline (speedup 1.0000x reference)
import functools

import jax
import jax.numpy as jnp
from jax import lax
from jax.experimental import pallas as pl
from jax.experimental.pallas import tpu as pltpu
from jax.experimental.pallas import tpu_sc as plsc

N = 10000
E = 160000
IN_C = 256
HID = 128
OUT_C = 3

NC = 2
NS = 16
L = 16
NPAD = 10240
DROWS = NPAD // L
EPS = E // NS
NW = NC * NS
EPW = E // NW
CHUNK = 128
REAL = 125
NCHUNK = EPW // REAL
RPS = N // NS
NACC = N + CHUNK

_sc_mesh = plsc.VectorSubcoreMesh(
    core_axis_name="c", subcore_axis_name="s", num_cores=NC, num_subcores=NS)


@functools.partial(
    pl.kernel,
    out_type=jax.ShapeDtypeStruct((NC, DROWS, L), jnp.float32),
    mesh=_sc_mesh,
    scratch_types=[
        pltpu.VMEM((EPS,), jnp.int32),
        pltpu.VMEM((DROWS, L), jnp.float32),
        pltpu.VMEM((DROWS // NS, L), jnp.float32),
        pltpu.VMEM((5, 128), jnp.int32),
        pltpu.VMEM_SHARED((DROWS, L), jnp.float32),
    ],
)
def _deg_dis_kernel(dst_hbm, out_hbm, dst_v, deg_v, tmp_v, rowidx_v, acc_sh):
    c = lax.axis_index("c")
    s = lax.axis_index("s")
    zeros16 = jnp.zeros((L,), jnp.float32)
    ones16 = jnp.ones((L,), jnp.float32)
    rps = DROWS // NS

    def zero_deg(t, carry):
        deg_v[t, :] = zeros16
        return carry
    lax.fori_loop(0, DROWS, zero_deg, 0)

    def zero_tmp(t, carry):
        tmp_v[t, :] = zeros16
        return carry
    lax.fori_loop(0, rps, zero_tmp, 0)

    def fill_rowidx(t, carry):
        k = t // 8
        j = t % 8
        rowidx_v[k, pl.ds(j * L, L)] = lax.iota(jnp.int32, (L,)) + k * 128 + j * L
        return carry
    lax.fori_loop(0, rps, fill_rowidx, 0)

    pltpu.sync_copy(tmp_v, acc_sh.at[pl.ds(s * rps, rps)])

    pltpu.sync_copy(dst_hbm.at[pl.ds(s * EPS, EPS)], dst_v)

    def hist(i, carry):
        idx = dst_v[pl.ds(i * L, L)]
        plsc.addupdate_scatter(deg_v, [idx >> 4, idx & 15], ones16)
        return carry
    lax.fori_loop(0, EPS // L, hist, 0)

    plsc.subcore_barrier()
    for k in range(5):
        pltpu.sync_copy(deg_v.at[pl.ds(k * 128, 128)],
                        acc_sh.at[rowidx_v.at[k]], add=True)
    plsc.subcore_barrier()

    pltpu.sync_copy(acc_sh.at[pl.ds(s * rps, rps)], deg_v.at[pl.ds(0, rps)])

    def rsq(t, carry):
        d = deg_v[t, :] + 1.0
        i = plsc.bitcast(d, jnp.int32)
        y = plsc.bitcast(jnp.int32(0x5F3759DF) - (i >> 1), jnp.float32)
        y = y * (1.5 - 0.5 * d * y * y)
        y = y * (1.5 - 0.5 * d * y * y)
        y = y * (1.5 - 0.5 * d * y * y)
        tmp_v[t, :] = y
        return carry
    lax.fori_loop(0, rps, rsq, 0)
    pltpu.sync_copy(tmp_v, out_hbm.at[c, pl.ds(s * rps, rps)])


def _make_agg_kernel(D):
    DL = D // L

    @functools.partial(
        pl.kernel,
        out_type=jax.ShapeDtypeStruct((NC, N, D), jnp.float32),
        mesh=_sc_mesh,
        scratch_types=[
            pltpu.VMEM((NCHUNK, CHUNK), jnp.int32),
            pltpu.VMEM((NCHUNK, CHUNK), jnp.int32),
            pltpu.VMEM((CHUNK, D), jnp.float32),
            pltpu.VMEM_SHARED((NACC, D), jnp.float32),
            pltpu.SemaphoreType.DMA,
        ],
    )
    def agg(g_hbm, src_hbm, dst_hbm, out_hbm, src_v, dst_v, rows_v, acc_sh, sem):
        c = lax.axis_index("c")
        s = lax.axis_index("s")
        w = c * NS + s
        zeros16 = jnp.zeros((L,), jnp.float32)

        def zero_rows(t, carry):
            rows_v[t // DL, pl.ds((t % DL) * L, L)] = zeros16
            return carry
        lax.fori_loop(0, CHUNK * DL, zero_rows, 0)

        for k in range(RPS // REAL):
            pltpu.sync_copy(rows_v.at[pl.ds(0, REAL)],
                            acc_sh.at[pl.ds(s * RPS + k * REAL, REAL)])

        pltpu.sync_copy(src_hbm.at[w], src_v)
        pltpu.sync_copy(dst_hbm.at[w], dst_v)
        plsc.subcore_barrier()

        def step(j, carry):
            pltpu.async_copy(g_hbm.at[src_v.at[j]], rows_v, sem).wait()
            pltpu.sync_copy(rows_v, acc_sh.at[dst_v.at[j]], add=True)
            return carry
        lax.fori_loop(0, NCHUNK, step, 0)

        plsc.subcore_barrier()
        for k in range(RPS // REAL):
            base = s * RPS + k * REAL
            pltpu.sync_copy(acc_sh.at[pl.ds(base, REAL)],
                            rows_v.at[pl.ds(0, REAL)])
            pltpu.sync_copy(rows_v.at[pl.ds(0, REAL)],
                            out_hbm.at[c, pl.ds(base, REAL)])
    return agg


_agg128 = _make_agg_kernel(HID)
_agg16 = _make_agg_kernel(16)


def _mm1_body(f_ref, w_ref, d_ref, o_ref):
    h = jnp.dot(f_ref[...], w_ref[...], preferred_element_type=jnp.float32)
    o_ref[...] = h * d_ref[...]


def _combine1_body(a_ref, g_ref, d_ref, b1_ref, w2_ref, o_ref):
    x = (a_ref[0] + a_ref[1] + g_ref[...]) * d_ref[...] + b1_ref[...]
    x = jnp.maximum(x, 0.0)
    o_ref[...] = jnp.dot(x, w2_ref[...],
                         preferred_element_type=jnp.float32) * d_ref[...]


def _combine2_body(a_ref, g_ref, d_ref, b2_ref, o_ref):
    o_ref[...] = (a_ref[0] + a_ref[1] + g_ref[...]) * d_ref[...] + b2_ref[...]


def kernel(features, edges, edges2, edge_features, additional_feature,
           W1, b1, W2, b2):
    src = edges[0]
    dst = edges[1]

    dis_t = _deg_dis_kernel(dst)
    dis = dis_t[0].reshape(NPAD)[:N].reshape(N, 1)

    g1 = pl.pallas_call(
        _mm1_body,
        out_shape=jax.ShapeDtypeStruct((N, HID), jnp.float32),
    )(features, W1, dis)

    src3 = src.reshape(NW, NCHUNK, REAL)
    dst3 = dst.reshape(NW, NCHUNK, REAL)
    pad_src = jnp.zeros((NW, NCHUNK, CHUNK - REAL), jnp.int32)
    pad_dst = jnp.full((NW, NCHUNK, CHUNK - REAL), N, jnp.int32)
    src3 = jnp.concatenate([src3, pad_src], axis=2)
    dst3 = jnp.concatenate([dst3, pad_dst], axis=2)

    agg1 = _agg128(g1, src3, dst3)

    W2p = jnp.zeros((HID, 16), jnp.float32).at[:, :OUT_C].set(W2)
    b1r = b1.reshape(1, HID)
    g2 = pl.pallas_call(
        _combine1_body,
        out_shape=jax.ShapeDtypeStruct((N, 16), jnp.float32),
    )(agg1, g1, dis, b1r, W2p)

    agg2 = _agg16(g2, src3, dst3)

    b2p = jnp.zeros((1, 16), jnp.float32).at[0, :OUT_C].set(b2)
    y = pl.pallas_call(
        _combine2_body,
        out_shape=jax.ShapeDtypeStruct((N, 16), jnp.float32),
    )(agg2, g2, dis, b2p)
    return y[:, :OUT_C]


# speedup vs baseline: 10.4336x; 10.4336x over previous
import functools

import jax
import jax.numpy as jnp
from jax import lax
from jax.experimental import pallas as pl
from jax.experimental.pallas import tpu as pltpu
from jax.experimental.pallas import tpu_sc as plsc

N = 10000
E = 160000
IN_C = 256
HID = 128
OUT_C = 3

NC = 2
NS = 16
L = 16
NPAD = 10240
DROWS = NPAD // L
EPS = E // NS
NW = NC * NS
EPW = E // NW
CHUNK = 128
REAL = 125
NCHUNK = EPW // REAL
SLAB = 624
ZCH = ((0, 128), (128, 128), (256, 128), (384, 128), (512, 112))
TAIL = N - NS * SLAB
NACC = N + CHUNK

_sc_mesh = plsc.VectorSubcoreMesh(
    core_axis_name="c", subcore_axis_name="s", num_cores=NC, num_subcores=NS)


@functools.partial(
    pl.kernel,
    out_type=jax.ShapeDtypeStruct((NC, DROWS, L), jnp.float32),
    mesh=_sc_mesh,
    scratch_types=[
        pltpu.VMEM((EPS,), jnp.int32),
        pltpu.VMEM((NPAD,), jnp.float32),
        pltpu.VMEM((DROWS, L), jnp.float32),
        pltpu.VMEM((DROWS // NS, L), jnp.float32),
        pltpu.VMEM((5, 128), jnp.int32),
        pltpu.VMEM_SHARED((DROWS, L), jnp.float32),
    ],
    compiler_params=pltpu.CompilerParams(needs_layout_passes=False),
)
def _deg_dis_kernel(dst_hbm, out_hbm, dst_v, deg1_v, deg_v, tmp_v, rowidx_v,
                    acc_sh):
    c = lax.axis_index("c")
    s = lax.axis_index("s")
    zeros16 = jnp.zeros((L,), jnp.float32)
    ones16 = jnp.ones((L,), jnp.float32)
    rps = DROWS // NS

    def zero_deg(t, carry):
        deg1_v[pl.ds(t * L, L)] = zeros16
        return carry
    lax.fori_loop(0, DROWS, zero_deg, 0)

    def zero_tmp(t, carry):
        tmp_v[t, :] = zeros16
        return carry
    lax.fori_loop(0, rps, zero_tmp, 0)

    def fill_rowidx(t, carry):
        k = t // 8
        j = t % 8
        rowidx_v[k, pl.ds(j * L, L)] = lax.iota(jnp.int32, L) + k * 128 + j * L
        return carry
    lax.fori_loop(0, rps, fill_rowidx, 0)

    pltpu.sync_copy(tmp_v, acc_sh.at[pl.ds(s * rps, rps)])

    pltpu.sync_copy(dst_hbm.at[s], dst_v)

    def hist(i, carry):
        idx = dst_v[pl.ds(i * L, L)]
        plsc.addupdate_scatter(deg1_v, [idx], ones16)
        return carry
    lax.fori_loop(0, EPS // L, hist, 0)

    def repack(t, carry):
        deg_v[t, :] = deg1_v[pl.ds(t * L, L)]
        return carry
    lax.fori_loop(0, DROWS, repack, 0)

    plsc.subcore_barrier()
    for k in range(5):
        pltpu.sync_copy(deg_v.at[pl.ds(k * 128, 128)],
                        acc_sh.at[rowidx_v.at[k]], add=True)
    plsc.subcore_barrier()

    pltpu.sync_copy(acc_sh.at[pl.ds(s * rps, rps)], deg_v.at[pl.ds(0, rps)])

    def rsq(t, carry):
        d = deg_v[t, :] + 1.0
        i = plsc.bitcast(d, jnp.int32)
        y = plsc.bitcast(jnp.int32(0x5F3759DF) - (i >> 1), jnp.float32)
        y = y * (1.5 - 0.5 * d * y * y)
        y = y * (1.5 - 0.5 * d * y * y)
        y = y * (1.5 - 0.5 * d * y * y)
        tmp_v[t, :] = y
        return carry
    lax.fori_loop(0, rps, rsq, 0)
    pltpu.sync_copy(tmp_v, out_hbm.at[c, pl.ds(s * rps, rps)])


def _make_agg_kernel(D):
    DL = D // L

    @functools.partial(
        pl.kernel,
        out_type=jax.ShapeDtypeStruct((NC, N, D), jnp.float32),
        mesh=_sc_mesh,
        scratch_types=[
            pltpu.VMEM((NCHUNK, CHUNK), jnp.int32),
            pltpu.VMEM((NCHUNK, CHUNK), jnp.int32),
            pltpu.VMEM((CHUNK, D), jnp.float32),
            pltpu.VMEM_SHARED((NACC, D), jnp.float32),
            pltpu.SemaphoreType.DMA,
        ],
        compiler_params=pltpu.CompilerParams(needs_layout_passes=False),
    )
    def agg(g_hbm, src_hbm, dst_hbm, out_hbm, src_v, dst_v, rows_v, acc_sh, sem):
        c = lax.axis_index("c")
        s = lax.axis_index("s")
        w = c * NS + s
        zeros16 = jnp.zeros((L,), jnp.float32)

        def zero_rows(t, carry):
            rows_v[t // DL, pl.ds((t % DL) * L, L)] = zeros16
            return carry
        lax.fori_loop(0, CHUNK * DL, zero_rows, 0)

        for off, ln in ZCH:
            pltpu.sync_copy(rows_v.at[pl.ds(0, ln)],
                            acc_sh.at[pl.ds(s * SLAB + off, ln)])

        @pl.when(s == 0)
        def _zero_tail():
            pltpu.sync_copy(rows_v.at[pl.ds(0, TAIL)],
                            acc_sh.at[pl.ds(NS * SLAB, TAIL)])

        pltpu.sync_copy(src_hbm.at[w], src_v)
        pltpu.sync_copy(dst_hbm.at[w], dst_v)
        plsc.subcore_barrier()

        def step(j, carry):
            pltpu.async_copy(g_hbm.at[src_v.at[j]], rows_v, sem).wait()
            pltpu.sync_copy(rows_v, acc_sh.at[dst_v.at[j]], add=True)
            return carry
        lax.fori_loop(0, NCHUNK, step, 0)

        plsc.subcore_barrier()
        for off, ln in ZCH:
            base = s * SLAB + off
            pltpu.sync_copy(acc_sh.at[pl.ds(base, ln)],
                            rows_v.at[pl.ds(0, ln)])
            pltpu.sync_copy(rows_v.at[pl.ds(0, ln)],
                            out_hbm.at[c, pl.ds(base, ln)])

        @pl.when(s == 0)
        def _out_tail():
            pltpu.sync_copy(acc_sh.at[pl.ds(NS * SLAB, TAIL)],
                            rows_v.at[pl.ds(0, TAIL)])
            pltpu.sync_copy(rows_v.at[pl.ds(0, TAIL)],
                            out_hbm.at[c, pl.ds(NS * SLAB, TAIL)])
    return agg


_agg128 = _make_agg_kernel(HID)


def _mm1_body(f_ref, w_ref, d_ref, o_ref):
    h = jnp.dot(f_ref[...], w_ref[...], preferred_element_type=jnp.float32)
    o_ref[...] = h * d_ref[...]


def _combine1_body(a_ref, g_ref, d_ref, b1_ref, o_ref):
    x = (a_ref[0] + a_ref[1] + g_ref[...]) * d_ref[...] + b1_ref[...]
    o_ref[...] = jnp.maximum(x, 0.0) * d_ref[...]


def _combine2_body(a_ref, y_ref, d_ref, b2_ref, w2_ref, o_ref):
    t = a_ref[0] + a_ref[1] + y_ref[...]
    o_ref[...] = jnp.dot(t, w2_ref[...],
                         preferred_element_type=jnp.float32) * d_ref[...] + b2_ref[...]


def kernel(features, edges, edges2, edge_features, additional_feature,
           W1, b1, W2, b2):
    src = edges[0]
    dst = edges[1]

    dis_t = _deg_dis_kernel(dst.reshape(NS, EPS))
    dis = dis_t[0].reshape(NPAD)[:N].reshape(N, 1)

    g1 = pl.pallas_call(
        _mm1_body,
        out_shape=jax.ShapeDtypeStruct((N, HID), jnp.float32),
    )(features, W1, dis)

    src3 = src.reshape(NW, NCHUNK, REAL)
    dst3 = dst.reshape(NW, NCHUNK, REAL)
    pad_src = jnp.zeros((NW, NCHUNK, CHUNK - REAL), jnp.int32)
    pad_dst = jnp.full((NW, NCHUNK, CHUNK - REAL), N, jnp.int32)
    src3 = jnp.concatenate([src3, pad_src], axis=2)
    dst3 = jnp.concatenate([dst3, pad_dst], axis=2)

    agg1 = _agg128(g1, src3, dst3)

    b1r = b1.reshape(1, HID)
    y = pl.pallas_call(
        _combine1_body,
        out_shape=jax.ShapeDtypeStruct((N, HID), jnp.float32),
    )(agg1, g1, dis, b1r)

    agg2 = _agg128(y, src3, dst3)

    W2p = jnp.zeros((HID, 16), jnp.float32).at[:, :OUT_C].set(W2)
    b2p = jnp.zeros((1, 16), jnp.float32).at[0, :OUT_C].set(b2)
    out = pl.pallas_call(
        _combine2_body,
        out_shape=jax.ShapeDtypeStruct((N, 16), jnp.float32),
    )(agg2, y, dis, b2p, W2p)
    return out[:, :OUT_C]
